# baseline (speedup 1.0000x reference)
import numpy as np
import jax
import jax.numpy as jnp
from jax.experimental import pallas as pl
from jax.experimental.pallas import tpu as pltpu

D_MODEL = 256
_B = 2048


def _body(g_ref, r_ref, t_ref, q_ref, pv_ref, hp_ref, w_ref,
          out_ref, sum_ref, wl_ref, wr_ref):
    i = pl.program_id(0)
    nb = pl.num_programs(0)
    B = out_ref.shape[0]

    @pl.when(i == 0)
    def _init():
        col = jax.lax.broadcasted_iota(jnp.int32, (64, 128), 1)
        coord = jax.lax.broadcasted_iota(jnp.int32, (64, 128), 0)
        freq = jnp.exp((col // 2).astype(jnp.float32)
                       * jnp.float32(-2.0 * np.log(10000.0) / 128.0))
        phase = (col % 2).astype(jnp.float32) * jnp.float32(np.pi / 2.0)
        pe = jnp.sin(coord.astype(jnp.float32) * freq + phase).astype(jnp.bfloat16)
        wl_ref[0:64, :] = w_ref[0:64, 0:128].astype(jnp.bfloat16)
        wl_ref[64:128, :] = pe
        wr_ref[0:64, :] = w_ref[0:64, 128:256].astype(jnp.bfloat16)
        wr_ref[64:128, :] = pe
        sum_ref[...] = jnp.zeros_like(sum_ref)

    col = jax.lax.broadcasted_iota(jnp.int32, (B, 128), 1)
    g = g_ref[0, 0, :].reshape(B, 1)
    r = r_ref[0, 0, :].reshape(B, 1)
    t = t_ref[0, 0, :].reshape(B, 1)
    q = q_ref[0, 0, :].reshape(B, 1)
    pv = pv_ref[0, 0, :].reshape(B, 1)
    hp = hp_ref[0, 0, :].reshape(B, 1)

    m1 = ((col == g).astype(jnp.float32)
          + (col - 64 == t).astype(jnp.float32)).astype(jnp.bfloat16)
    m2 = ((col == r).astype(jnp.float32)
          + (col - 64 == q).astype(jnp.float32)
          + jnp.where(col == 4, pv, 0.0)
          + jnp.where(col == 5, hp, 0.0)
          + (col == 6).astype(jnp.float32)).astype(jnp.bfloat16)

    bl = jnp.dot(m1, wl_ref[...], preferred_element_type=jnp.float32)
    br = jnp.dot(m2, wr_ref[...], preferred_element_type=jnp.float32)
    out_ref[:, 0:128] = bl
    out_ref[:, 128:256] = br
    sum_ref[0:1, 0:128] += jnp.sum(bl, axis=0, keepdims=True)
    sum_ref[0:1, 128:256] += jnp.sum(br, axis=0, keepdims=True)

    @pl.when(i == nb - 1)
    def _fin():
        sum_ref[...] *= jnp.float32(1.0 / (nb * B))


def kernel(gate_idx, role_idx, param_val, has_param, grid_positions,
           gate_table, role_table, W_param, b_param):
    N = gate_idx.shape[0]
    nb = N // _B

    w_all = jnp.zeros((64, D_MODEL), jnp.float32)
    w_all = w_all.at[0:64, 0:128].set(gate_table)
    w_all = w_all.at[0:4, 128:192].set(role_table)
    w_all = w_all.at[4, 192:255].set(W_param[0])
    w_all = w_all.at[5, 255].set(1.0)
    w_all = w_all.at[6, 192:255].set(b_param)

    def shp(a):
        return a.reshape(nb, 1, _B)

    g3 = shp(gate_idx.astype(jnp.int32))
    r3 = shp(role_idx.astype(jnp.int32))
    t3 = shp(grid_positions[:, 0].astype(jnp.int32))
    q3 = shp(grid_positions[:, 1].astype(jnp.int32))
    pv3 = shp(param_val)
    hp3 = shp(has_param)

    idx_spec = pl.BlockSpec((1, 1, _B), lambda i: (i, 0, 0))
    rep_spec_w = pl.BlockSpec((64, D_MODEL), lambda i: (0, 0))

    out, ssum = pl.pallas_call(
        _body,
        grid=(nb,),
        in_specs=[idx_spec, idx_spec, idx_spec, idx_spec, idx_spec, idx_spec,
                  rep_spec_w],
        out_specs=[pl.BlockSpec((_B, D_MODEL), lambda i: (i, 0)),
                   pl.BlockSpec((1, D_MODEL), lambda i: (0, 0))],
        out_shape=[jax.ShapeDtypeStruct((N, D_MODEL), jnp.float32),
                   jax.ShapeDtypeStruct((1, D_MODEL), jnp.float32)],
        scratch_shapes=[pltpu.VMEM((128, 128), jnp.bfloat16),
                        pltpu.VMEM((128, 128), jnp.bfloat16)],
    )(g3, r3, t3, q3, pv3, hp3, w_all)

    return out, ssum.reshape(D_MODEL)
